# baseline (speedup 1.0000x reference)
import functools

import jax
import jax.numpy as jnp
from jax import lax
from jax.experimental import pallas as pl
from jax.experimental.pallas import tpu as pltpu
from jax.experimental.pallas import tpu_sc as plsc

VOCAB = 1000000
D = 64
DP = 128
SCALE = 8.0

NC, NS, L = 2, 16, 16
NW = NC * NS

RP_BLK = 8000

XROWS = 4096
XCOLS = 200
XR_PER_W = XROWS // NW
G_ROWS = 40
G_PER_ROW = XCOLS // G_ROWS
QH = 16
NQ = XR_PER_W // QH

_mesh = plsc.VectorSubcoreMesh(core_axis_name="c", subcore_axis_name="s")


def _repack_body(t_ref, o_ref):
    blk = t_ref[...] * SCALE
    pad = jnp.zeros((RP_BLK, DP - D), jnp.float32)
    o_ref[...] = jnp.concatenate([blk, pad], axis=1)


_repack = pl.pallas_call(
    _repack_body,
    grid=(VOCAB // RP_BLK,),
    in_specs=[pl.BlockSpec((RP_BLK, D), lambda i: (i, 0))],
    out_specs=pl.BlockSpec((RP_BLK, DP), lambda i: (i, 0)),
    out_shape=jax.ShapeDtypeStruct((VOCAB, DP), jnp.float32),
)


@functools.partial(
    pl.kernel,
    out_type=jax.ShapeDtypeStruct((XROWS, XCOLS, D), jnp.float32),
    mesh=_mesh,
    scratch_types=[
        pltpu.VMEM((QH, G_PER_ROW, G_ROWS), jnp.int32),
        pltpu.VMEM((2, XCOLS, DP), jnp.float32),
        pltpu.VMEM((2, XCOLS, D), jnp.float32),
        pltpu.SemaphoreType.DMA,
        pltpu.SemaphoreType.DMA,
        pltpu.SemaphoreType.DMA,
        pltpu.SemaphoreType.DMA,
    ],
)
def _gather(x_hbm, wide_hbm, out_hbm, idx_v, gbuf, obuf, gsem0, gsem1, ssem0, ssem1):
    wid = lax.axis_index("s") * NC + lax.axis_index("c")
    xr_base = wid * XR_PER_W
    gsems = (gsem0, gsem1)
    ssems = (ssem0, ssem1)

    def issue_chunk(c, b):
        for j in range(G_PER_ROW):
            pltpu.async_copy(
                wide_hbm.at[idx_v.at[c, j, :]],
                gbuf.at[b, pl.ds(j * G_ROWS, G_ROWS), :],
                gsems[b],
            )

    def wait_chunk(b):
        for j in range(G_PER_ROW):
            pltpu.make_async_copy(
                wide_hbm.at[pl.ds(0, G_ROWS)],
                gbuf.at[b, pl.ds(j * G_ROWS, G_ROWS), :],
                gsems[b],
            ).wait()

    def wait_store(b):
        pltpu.make_async_copy(out_hbm.at[0], obuf.at[b], ssems[b]).wait()

    def compact_chunk(b):
        def body(r, carry):
            for j in range(D // L):
                obuf[b, r, pl.ds(j * L, L)] = gbuf[b, r, pl.ds(j * L, L)]
            return carry

        lax.fori_loop(0, XCOLS, body, 0)

    for q in range(NQ):
        hr0 = xr_base + q * QH
        pltpu.sync_copy(x_hbm.at[pl.ds(hr0, QH), :, :], idx_v)
        issue_chunk(0, 0)

        def super_body(s, carry):
            c0 = s * 2
            for b in range(2):
                c = c0 + b

                @pl.when(c + 1 < QH)
                def _():
                    issue_chunk(c + 1, 1 - b)

                wait_chunk(b)

                @pl.when(c >= 2)
                def _():
                    wait_store(b)

                compact_chunk(b)
                pltpu.async_copy(obuf.at[b], out_hbm.at[hr0 + c], ssems[b])
            return carry

        lax.fori_loop(0, QH // 2, super_body, 0)
        wait_store(0)
        wait_store(1)


def kernel(x, table):
    x4 = x.astype(jnp.int32).reshape(XROWS, G_PER_ROW, G_ROWS)
    wide = _repack(table)
    return _gather(x4, wide)
